# baseline (speedup 1.0000x reference)
import jax
import jax.numpy as jnp
from jax.experimental import pallas as pl
from jax.experimental.pallas import tpu as pltpu

_B, _L, _CL, _D, _E, _G = 32, 64, 512, 1024, 8, 2
_F = 3 * _CL
_R = _B * _L
_NE = _E + _G
_EPS = 1e-9

_DB = 512
_ND = _D // _DB
_RB = 256
_NR = _R // _RB
_NSUB = _NE + _NR


def _moe_kernel(logits_ref, mask_ref, flat_ref, we_ref, wg_ref, b_ref,
                out_ref, wscr_ref, fbf_ref):
    dh = pl.program_id(0)
    sub = pl.program_id(1)

    @pl.when((dh == 0) & (sub == 0))
    def _cast_flat():
        fbf_ref[...] = flat_ref[...].astype(jnp.bfloat16)

    @pl.when(sub < _E)
    def _cast_expert_slab():
        wscr_ref[sub] = we_ref[0].astype(jnp.bfloat16)

    @pl.when((sub >= _E) & (sub < _NE))
    def _cast_general_slab():
        wscr_ref[sub] = wg_ref[0].astype(jnp.bfloat16)

    @pl.when(sub >= _NE)
    def _compute():
        r = sub - _NE

        logits = logits_ref[...]
        maskf = jnp.where(mask_ref[...] == 1, 1.0, 0.0).astype(jnp.float32)
        g = jax.nn.softmax(logits, axis=1) * maskf
        g = g / (jnp.sum(g, axis=1, keepdims=True) + _EPS)

        rowb = (jax.lax.broadcasted_iota(jnp.int32, (_RB, _B), 0)
                + r * _RB) // _L
        blane = jax.lax.broadcasted_iota(jnp.int32, (_RB, _B), 1)
        onehot = (rowb == blane).astype(jnp.float32)
        grow = jnp.dot(onehot, g, preferred_element_type=jnp.float32)

        fbf = fbf_ref[pl.ds(r * _RB, _RB), :]

        acc = jnp.dot(grow, b_ref[:_E, :], preferred_element_type=jnp.float32)
        for e in range(_E):
            y = jnp.dot(fbf, wscr_ref[e], preferred_element_type=jnp.float32)
            acc += grow[:, e:e + 1] * y
        acc = acc.astype(jnp.bfloat16).astype(jnp.float32)
        for i in range(_E, _NE):
            acc += jnp.dot(fbf, wscr_ref[i], preferred_element_type=jnp.float32)
            acc += b_ref[i:i + 1, :]
        out_ref[...] = acc


def kernel(cycle_curve_data, logits, moe_masks, We, be, Wg, bg):
    flat = cycle_curve_data.reshape(_R, _F)
    b_all = jnp.zeros((16, _D), jnp.float32)
    b_all = b_all.at[:_E].set(be).at[_E:_NE].set(bg)

    out = pl.pallas_call(
        _moe_kernel,
        grid=(_ND, _NSUB),
        in_specs=[
            pl.BlockSpec((_B, _E), lambda d, s: (0, 0)),
            pl.BlockSpec((_B, _E), lambda d, s: (0, 0)),
            pl.BlockSpec((_R, _F), lambda d, s: (0, 0)),
            pl.BlockSpec((1, _F, _DB),
                         lambda d, s: (jnp.clip(s, 0, _E - 1), 0, d)),
            pl.BlockSpec((1, _F, _DB),
                         lambda d, s: (jnp.clip(s - _E, 0, _G - 1), 0, d)),
            pl.BlockSpec((16, _DB), lambda d, s: (0, d)),
        ],
        out_specs=pl.BlockSpec(
            (_RB, _DB), lambda d, s: (jnp.maximum(s - _NE, 0), d)),
        out_shape=jax.ShapeDtypeStruct((_R, _D), jnp.float32),
        scratch_shapes=[
            pltpu.VMEM((_NE, _F, _DB), jnp.bfloat16),
            pltpu.VMEM((_R, _F), jnp.bfloat16),
        ],
    )(logits, moe_masks.astype(jnp.int32), flat, We, Wg, b_all)

    final_out = out.reshape(_B, _L, _D)
    aug_loss = jnp.zeros((), dtype=jnp.float32)
    guide_loss = jnp.zeros((), dtype=jnp.float32)
    return (final_out, aug_loss, guide_loss)
